# baseline (speedup 1.0000x reference)
import functools

import jax
import jax.numpy as jnp
from jax import lax
from jax.experimental import pallas as pl
from jax.experimental.pallas import tpu as pltpu
from jax.experimental.pallas import tpu_sc as plsc

NUM_EMBEDDINGS = 1000000
DETILE_COLS = 8192
EMBEDDING_DIM = 64
BT = 128
DT = 8
TPU_ = 2
BURST = 8


@functools.cache
def _build(nb: int, ns_seq: int):
    info = plsc.get_sparse_core_info()
    nc, nsub = info.num_cores, info.num_subcores
    nw = nc * nsub
    assert nb % TPU_ == 0
    nu = nb // TPU_
    n_units = ns_seq * nu
    assert n_units % nw == 0
    per_worker = n_units // nw
    assert per_worker % 2 == 0 and per_worker >= 4
    n_dtiles = EMBEDDING_DIM // DT
    rows_per_unit = TPU_ * BT
    unit_cols = TPU_ * DT * BT

    mesh = plsc.VectorSubcoreMesh(core_axis_name="c", subcore_axis_name="s")

    @functools.partial(
        pl.kernel,
        mesh=mesh,
        out_type=jax.ShapeDtypeStruct(
            (ns_seq, n_dtiles, nb, DT, BT), jnp.float32),
        scratch_types=[
            [pltpu.VMEM((rows_per_unit,), jnp.int32)] * 2,
            [pltpu.VMEM((rows_per_unit, EMBEDDING_DIM), jnp.float32)] * 2,
            [pltpu.VMEM((n_dtiles, TPU_, DT, BT), jnp.float32)] * 2,
            [pltpu.SemaphoreType.DMA] * 2,
            [pltpu.SemaphoreType.DMA] * 2,
            [pltpu.SemaphoreType.DMA] * 2,
        ],
        compiler_params=pltpu.CompilerParams(
            use_tc_tiling_on_sc=False,
            needs_layout_passes=False,
            disable_bounds_checks=True,
        ),
    )
    def gather_kernel(w4_hbm, table_hbm, out_hbm,
                      idx_v, rows, planes, isem, gsem, ssem):
        wid = lax.axis_index("s") * nc + lax.axis_index("c")
        u0 = wid * per_worker

        def unit_st(j):
            u = u0 + j
            return u // nu, (u % nu) * TPU_

        def start_idx(b, j):
            s, tj = unit_st(j)
            for t in range(TPU_):
                pltpu.async_copy(w4_hbm.at[s // DT, tj + t, s % DT],
                                 idx_v[b].at[pl.ds(t * BT, BT)], isem[b])

        def wait_idx(b, j):
            s, tj = unit_st(j)
            for t in range(TPU_):
                pltpu.make_async_copy(w4_hbm.at[s // DT, tj + t, s % DT],
                                      idx_v[b].at[pl.ds(t * BT, BT)],
                                      isem[b]).wait()

        def start_gather(b):
            pltpu.async_copy(table_hbm.at[idx_v[b]], rows[b], gsem[b])

        def wait_gather(b):
            pltpu.make_async_copy(table_hbm.at[idx_v[b]], rows[b],
                                  gsem[b]).wait()

        def out_slice(j):
            s, tj = unit_st(j)
            return out_hbm.at[s, :, pl.ds(tj, TPU_)]

        def start_store(b, j):
            pltpu.async_copy(planes[b], out_slice(j), ssem[b])

        def wait_store(b, j):
            pltpu.make_async_copy(planes[b], out_slice(j), ssem[b]).wait()

        lanes16 = lax.iota(jnp.int32, 16)
        n_db = EMBEDDING_DIM // 16
        mvec = [(lanes16 + k) % 16 for k in range(16)]
        colc = [[mvec[k] + db * 16 for db in range(n_db)] for k in range(16)]
        tivc = [[mvec[k] // DT + db * 2 for db in range(n_db)]
                for k in range(16)]
        divc = [mvec[k] % DT for k in range(16)]

        def transpose(b):
            src = rows[b]
            dst = planes[b]

            @plsc.parallel_loop(0, rows_per_unit // 16)
            def _(rb):
                r0 = rb * 16
                t = r0 // BT
                bi0 = r0 % BT
                tvec = jnp.zeros((16,), jnp.int32) + t
                rvec = lanes16 + r0
                bvec = lanes16 + bi0
                for db in range(n_db):
                    for k in range(16):
                        v = plsc.load_gather(src, [rvec, colc[k][db]])
                        plsc.store_scatter(
                            dst, [tivc[k][db], tvec, divc[k], bvec], v)

        start_idx(0, 0)
        wait_idx(0, 0)
        start_gather(0)
        start_idx(1, 1)

        @pl.loop(0, per_worker // 2 - 1)
        def _(p):
            for b in range(2):
                j = 2 * p + b
                b1 = 1 - b
                wait_idx(b1, j + 1)
                start_gather(b1)
                wait_gather(b)
                start_idx(b, j + 2)
                pl.when(p >= 1)(lambda: wait_store(b, j - 2))
                transpose(b)
                start_store(b, j)

        j = per_worker - 2
        wait_idx(1, j + 1)
        start_gather(1)
        wait_gather(0)
        wait_store(0, j - 2)
        transpose(0)
        start_store(0, j)
        wait_gather(1)
        wait_store(1, j - 1)
        transpose(1)
        start_store(1, j + 1)
        wait_store(0, j)
        wait_store(1, j + 1)

    return gather_kernel


@functools.cache
def _build_detile(v: int, d: int):
    cols = DETILE_COLS
    grid = -(-v // cols)

    def body(in_ref, out_ref):
        t = in_ref[...].T
        out_ref[...] = jnp.concatenate([t[: cols // 2], t[cols // 2 :]],
                                       axis=1)

    return pl.pallas_call(
        body,
        grid=(grid,),
        in_specs=[pl.BlockSpec((d, cols), lambda j: (0, j))],
        out_specs=pl.BlockSpec((cols // 2, 2 * d), lambda j: (j, 0)),
        out_shape=jax.ShapeDtypeStruct((grid * cols // 2, 2 * d),
                                       jnp.float32),
        compiler_params=pltpu.CompilerParams(
            dimension_semantics=("arbitrary",)),
    )


def kernel(words, weight):
    nb4, ns_seq = words.shape
    nb = nb4 // BT
    w = words.astype(jnp.int32)
    c, h = DETILE_COLS, DETILE_COLS // 2
    q = (((w // c) * h + (w % h)) << 1) | ((w % c) // h)
    w4 = (q.reshape(nb, BT, ns_seq // DT, DT)
          .transpose(2, 0, 3, 1))
    inter = _build_detile(NUM_EMBEDDINGS, EMBEDDING_DIM)(weight.T)
    table = inter.reshape(-1, EMBEDDING_DIM)
    out5 = _build(nb, ns_seq)(w4, table)
    out = out5.transpose(2, 4, 0, 1, 3).reshape(nb4, ns_seq, EMBEDDING_DIM)
    return out


# speedup vs baseline: 1.0666x; 1.0666x over previous
import functools

import jax
import jax.numpy as jnp
from jax import lax
from jax.experimental import pallas as pl
from jax.experimental.pallas import tpu as pltpu
from jax.experimental.pallas import tpu_sc as plsc

NUM_EMBEDDINGS = 1000000
DETILE_COLS = 16384
EMBEDDING_DIM = 64
BT = 128
DT = 8
TPU_ = 2
BURST = 8


@functools.cache
def _build(nb: int, ns_seq: int):
    info = plsc.get_sparse_core_info()
    nc, nsub = info.num_cores, info.num_subcores
    nw = nc * nsub
    assert nb % TPU_ == 0
    nu = nb // TPU_
    n_units = ns_seq * nu
    assert n_units % nw == 0
    per_worker = n_units // nw
    assert per_worker % 2 == 0 and per_worker >= 4
    n_dtiles = EMBEDDING_DIM // DT
    rows_per_unit = TPU_ * BT
    unit_cols = TPU_ * DT * BT

    mesh = plsc.VectorSubcoreMesh(core_axis_name="c", subcore_axis_name="s")

    @functools.partial(
        pl.kernel,
        mesh=mesh,
        out_type=jax.ShapeDtypeStruct(
            (ns_seq, n_dtiles, nb, DT, BT), jnp.float32),
        scratch_types=[
            [pltpu.VMEM((rows_per_unit,), jnp.int32)] * 2,
            [pltpu.VMEM((rows_per_unit, EMBEDDING_DIM), jnp.float32)] * 2,
            [pltpu.VMEM((n_dtiles, TPU_, DT, BT), jnp.float32)] * 2,
            [pltpu.SemaphoreType.DMA] * 2,
            [pltpu.SemaphoreType.DMA] * 2,
            [pltpu.SemaphoreType.DMA] * 2,
        ],
        compiler_params=pltpu.CompilerParams(
            use_tc_tiling_on_sc=False,
            needs_layout_passes=False,
            disable_bounds_checks=True,
        ),
    )
    def gather_kernel(w4_hbm, table_hbm, out_hbm,
                      idx_v, rows, planes, isem, gsem, ssem):
        wid = lax.axis_index("s") * nc + lax.axis_index("c")
        u0 = wid * per_worker

        def unit_st(j):
            u = u0 + j
            return u // nu, (u % nu) * TPU_

        def start_idx(b, j):
            s, tj = unit_st(j)
            for t in range(TPU_):
                pltpu.async_copy(w4_hbm.at[s // DT, tj + t, s % DT],
                                 idx_v[b].at[pl.ds(t * BT, BT)], isem[b])

        def wait_idx(b, j):
            s, tj = unit_st(j)
            for t in range(TPU_):
                pltpu.make_async_copy(w4_hbm.at[s // DT, tj + t, s % DT],
                                      idx_v[b].at[pl.ds(t * BT, BT)],
                                      isem[b]).wait()

        def start_gather(b):
            pltpu.async_copy(table_hbm.at[idx_v[b]], rows[b], gsem[b])

        def wait_gather(b):
            pltpu.make_async_copy(table_hbm.at[idx_v[b]], rows[b],
                                  gsem[b]).wait()

        def out_slice(j):
            s, tj = unit_st(j)
            return out_hbm.at[s, :, pl.ds(tj, TPU_)]

        def start_store(b, j):
            pltpu.async_copy(planes[b], out_slice(j), ssem[b])

        def wait_store(b, j):
            pltpu.make_async_copy(planes[b], out_slice(j), ssem[b]).wait()

        lanes16 = lax.iota(jnp.int32, 16)
        n_db = EMBEDDING_DIM // 16
        mvec = [(lanes16 + k) % 16 for k in range(16)]
        colc = [[mvec[k] + db * 16 for db in range(n_db)] for k in range(16)]
        tivc = [[mvec[k] // DT + db * 2 for db in range(n_db)]
                for k in range(16)]
        divc = [mvec[k] % DT for k in range(16)]

        def transpose(b):
            src = rows[b]
            dst = planes[b]

            @plsc.parallel_loop(0, rows_per_unit // 16)
            def _(rb):
                r0 = rb * 16
                t = r0 // BT
                bi0 = r0 % BT
                tvec = jnp.zeros((16,), jnp.int32) + t
                rvec = lanes16 + r0
                bvec = lanes16 + bi0
                for db in range(n_db):
                    for k in range(16):
                        v = plsc.load_gather(src, [rvec, colc[k][db]])
                        plsc.store_scatter(
                            dst, [tivc[k][db], tvec, divc[k], bvec], v)

        start_idx(0, 0)
        wait_idx(0, 0)
        start_gather(0)
        start_idx(1, 1)

        @pl.loop(0, per_worker // 2 - 1)
        def _(p):
            for b in range(2):
                j = 2 * p + b
                b1 = 1 - b
                wait_idx(b1, j + 1)
                start_gather(b1)
                wait_gather(b)
                start_idx(b, j + 2)
                pl.when(p >= 1)(lambda: wait_store(b, j - 2))
                transpose(b)
                start_store(b, j)

        j = per_worker - 2
        wait_idx(1, j + 1)
        start_gather(1)
        wait_gather(0)
        wait_store(0, j - 2)
        transpose(0)
        start_store(0, j)
        wait_gather(1)
        wait_store(1, j - 1)
        transpose(1)
        start_store(1, j + 1)
        wait_store(0, j)
        wait_store(1, j + 1)

    return gather_kernel


@functools.cache
def _build_detile(v: int, d: int):
    cols = DETILE_COLS
    grid = -(-v // cols)

    def body(in_ref, out_ref):
        t = in_ref[...].T
        out_ref[...] = jnp.concatenate([t[: cols // 2], t[cols // 2 :]],
                                       axis=1)

    return pl.pallas_call(
        body,
        grid=(grid,),
        in_specs=[pl.BlockSpec((d, cols), lambda j: (0, j))],
        out_specs=pl.BlockSpec((cols // 2, 2 * d), lambda j: (j, 0)),
        out_shape=jax.ShapeDtypeStruct((grid * cols // 2, 2 * d),
                                       jnp.float32),
        compiler_params=pltpu.CompilerParams(
            dimension_semantics=("arbitrary",)),
    )


def kernel(words, weight):
    nb4, ns_seq = words.shape
    nb = nb4 // BT
    w = words.astype(jnp.int32)
    c, h = DETILE_COLS, DETILE_COLS // 2
    q = (((w // c) * h + (w % h)) << 1) | ((w % c) // h)
    w4 = (q.reshape(nb, BT, ns_seq // DT, DT)
          .transpose(2, 0, 3, 1))
    inter = _build_detile(NUM_EMBEDDINGS, EMBEDDING_DIM)(weight.T)
    table = inter.reshape(-1, EMBEDDING_DIM)
    out5 = _build(nb, ns_seq)(w4, table)
    out = out5.transpose(2, 4, 0, 1, 3).reshape(nb4, ns_seq, EMBEDDING_DIM)
    return out


# speedup vs baseline: 1.0927x; 1.0245x over previous
import functools

import jax
import jax.numpy as jnp
from jax import lax
from jax.experimental import pallas as pl
from jax.experimental.pallas import tpu as pltpu
from jax.experimental.pallas import tpu_sc as plsc

NUM_EMBEDDINGS = 1000000
DETILE_COLS = 32768
EMBEDDING_DIM = 64
BT = 128
DT = 8
TPU_ = 2
BURST = 8


@functools.cache
def _build(nb: int, ns_seq: int):
    info = plsc.get_sparse_core_info()
    nc, nsub = info.num_cores, info.num_subcores
    nw = nc * nsub
    assert nb % TPU_ == 0
    nu = nb // TPU_
    n_units = ns_seq * nu
    assert n_units % nw == 0
    per_worker = n_units // nw
    assert per_worker % 2 == 0 and per_worker >= 4
    n_dtiles = EMBEDDING_DIM // DT
    rows_per_unit = TPU_ * BT
    unit_cols = TPU_ * DT * BT

    mesh = plsc.VectorSubcoreMesh(core_axis_name="c", subcore_axis_name="s")

    @functools.partial(
        pl.kernel,
        mesh=mesh,
        out_type=jax.ShapeDtypeStruct(
            (ns_seq, n_dtiles, nb, DT, BT), jnp.float32),
        scratch_types=[
            [pltpu.VMEM((rows_per_unit,), jnp.int32)] * 2,
            [pltpu.VMEM((rows_per_unit, EMBEDDING_DIM), jnp.float32)] * 2,
            [pltpu.VMEM((n_dtiles, TPU_, DT, BT), jnp.float32)] * 2,
            [pltpu.SemaphoreType.DMA] * 2,
            [pltpu.SemaphoreType.DMA] * 2,
            [pltpu.SemaphoreType.DMA] * 2,
        ],
        compiler_params=pltpu.CompilerParams(
            use_tc_tiling_on_sc=False,
            needs_layout_passes=False,
            disable_bounds_checks=True,
        ),
    )
    def gather_kernel(w4_hbm, table_hbm, out_hbm,
                      idx_v, rows, planes, isem, gsem, ssem):
        wid = lax.axis_index("s") * nc + lax.axis_index("c")
        u0 = wid * per_worker

        def unit_st(j):
            u = u0 + j
            return u // nu, (u % nu) * TPU_

        def start_idx(b, j):
            s, tj = unit_st(j)
            for t in range(TPU_):
                pltpu.async_copy(w4_hbm.at[s // DT, tj + t, s % DT],
                                 idx_v[b].at[pl.ds(t * BT, BT)], isem[b])

        def wait_idx(b, j):
            s, tj = unit_st(j)
            for t in range(TPU_):
                pltpu.make_async_copy(w4_hbm.at[s // DT, tj + t, s % DT],
                                      idx_v[b].at[pl.ds(t * BT, BT)],
                                      isem[b]).wait()

        def start_gather(b):
            pltpu.async_copy(table_hbm.at[idx_v[b]], rows[b], gsem[b])

        def wait_gather(b):
            pltpu.make_async_copy(table_hbm.at[idx_v[b]], rows[b],
                                  gsem[b]).wait()

        def out_slice(j):
            s, tj = unit_st(j)
            return out_hbm.at[s, :, pl.ds(tj, TPU_)]

        def start_store(b, j):
            pltpu.async_copy(planes[b], out_slice(j), ssem[b])

        def wait_store(b, j):
            pltpu.make_async_copy(planes[b], out_slice(j), ssem[b]).wait()

        lanes16 = lax.iota(jnp.int32, 16)
        n_db = EMBEDDING_DIM // 16
        mvec = [(lanes16 + k) % 16 for k in range(16)]
        colc = [[mvec[k] + db * 16 for db in range(n_db)] for k in range(16)]
        tivc = [[mvec[k] // DT + db * 2 for db in range(n_db)]
                for k in range(16)]
        divc = [mvec[k] % DT for k in range(16)]

        def transpose(b):
            src = rows[b]
            dst = planes[b]

            @plsc.parallel_loop(0, rows_per_unit // 16)
            def _(rb):
                r0 = rb * 16
                t = r0 // BT
                bi0 = r0 % BT
                tvec = jnp.zeros((16,), jnp.int32) + t
                rvec = lanes16 + r0
                bvec = lanes16 + bi0
                for db in range(n_db):
                    for k in range(16):
                        v = plsc.load_gather(src, [rvec, colc[k][db]])
                        plsc.store_scatter(
                            dst, [tivc[k][db], tvec, divc[k], bvec], v)

        start_idx(0, 0)
        wait_idx(0, 0)
        start_gather(0)
        start_idx(1, 1)

        @pl.loop(0, per_worker // 2 - 1)
        def _(p):
            for b in range(2):
                j = 2 * p + b
                b1 = 1 - b
                wait_idx(b1, j + 1)
                start_gather(b1)
                wait_gather(b)
                start_idx(b, j + 2)
                pl.when(p >= 1)(lambda: wait_store(b, j - 2))
                transpose(b)
                start_store(b, j)

        j = per_worker - 2
        wait_idx(1, j + 1)
        start_gather(1)
        wait_gather(0)
        wait_store(0, j - 2)
        transpose(0)
        start_store(0, j)
        wait_gather(1)
        wait_store(1, j - 1)
        transpose(1)
        start_store(1, j + 1)
        wait_store(0, j)
        wait_store(1, j + 1)

    return gather_kernel


@functools.cache
def _build_detile(v: int, d: int):
    cols = DETILE_COLS
    grid = -(-v // cols)

    def body(in_ref, out_ref):
        t = in_ref[...].T
        out_ref[...] = jnp.concatenate([t[: cols // 2], t[cols // 2 :]],
                                       axis=1)

    return pl.pallas_call(
        body,
        grid=(grid,),
        in_specs=[pl.BlockSpec((d, cols), lambda j: (0, j))],
        out_specs=pl.BlockSpec((cols // 2, 2 * d), lambda j: (j, 0)),
        out_shape=jax.ShapeDtypeStruct((grid * cols // 2, 2 * d),
                                       jnp.float32),
        compiler_params=pltpu.CompilerParams(
            dimension_semantics=("arbitrary",)),
    )


def kernel(words, weight):
    nb4, ns_seq = words.shape
    nb = nb4 // BT
    w = words.astype(jnp.int32)
    c, h = DETILE_COLS, DETILE_COLS // 2
    q = (((w // c) * h + (w % h)) << 1) | ((w % c) // h)
    w4 = (q.reshape(nb, BT, ns_seq // DT, DT)
          .transpose(2, 0, 3, 1))
    inter = _build_detile(NUM_EMBEDDINGS, EMBEDDING_DIM)(weight.T)
    table = inter.reshape(-1, EMBEDDING_DIM)
    out5 = _build(nb, ns_seq)(w4, table)
    out = out5.transpose(2, 4, 0, 1, 3).reshape(nb4, ns_seq, EMBEDDING_DIM)
    return out
